# baseline (speedup 1.0000x reference)
import jax
import jax.numpy as jnp
from jax.experimental import pallas as pl
from jax.experimental.pallas import tpu as pltpu

_N_STATES = 4
_N_ACTIONS = 2
_HIDDEN_PAD = 128
_BLOCK_B = 4096


def _fused_mlp_kernel(x_ref, w1t_ref, b1r_ref, w2t_ref, b2r_ref, o_ref):
    h = jnp.maximum(
        jnp.dot(x_ref[...], w1t_ref[...], preferred_element_type=jnp.float32)
        + b1r_ref[...],
        0.0,
    )
    o_ref[...] = (
        jnp.dot(h, w2t_ref[...], preferred_element_type=jnp.float32)
        + b2r_ref[...]
    )


def kernel(x, w1p, b1p, w2p, b2p):
    B = x.shape[0]
    w1t = jnp.transpose(w1p)
    b1r = jnp.transpose(b1p)
    w2t = jnp.transpose(w2p[:_N_ACTIONS, :])
    b2r = jnp.transpose(b2p[:_N_ACTIONS, :])

    Bp = pl.cdiv(B, _BLOCK_B) * _BLOCK_B
    if Bp != B:
        x = jnp.pad(x, ((0, Bp - B), (0, 0)))
    num_blocks = Bp // _BLOCK_B

    out = pl.pallas_call(
        _fused_mlp_kernel,
        out_shape=jax.ShapeDtypeStruct((Bp, _N_ACTIONS), jnp.float32),
        grid=(num_blocks,),
        in_specs=[
            pl.BlockSpec((_BLOCK_B, _N_STATES), lambda i: (i, 0)),
            pl.BlockSpec((_N_STATES, _HIDDEN_PAD), lambda i: (0, 0)),
            pl.BlockSpec((1, _HIDDEN_PAD), lambda i: (0, 0)),
            pl.BlockSpec((_HIDDEN_PAD, _N_ACTIONS), lambda i: (0, 0)),
            pl.BlockSpec((1, _N_ACTIONS), lambda i: (0, 0)),
        ],
        out_specs=pl.BlockSpec((_BLOCK_B, _N_ACTIONS), lambda i: (i, 0)),
        compiler_params=pltpu.CompilerParams(
            dimension_semantics=("parallel",)),
    )(x, w1t, b1r, w2t, b2r)
    return out[:B]
